# Initial kernel scaffold; baseline (speedup 1.0000x reference)
import jax
import jax.numpy as jnp
from jax.experimental import pallas as pl


def kernel(inputs, table):
    raise NotImplementedError("write your pallas kernel here")


# speedup vs baseline: 11.7354x; 11.7354x over previous
import functools

import jax
import jax.numpy as jnp
import numpy as np
from jax import lax
from jax.experimental import pallas as pl
from jax.experimental.pallas import tpu as pltpu
from jax.experimental.pallas import tpu_sc as plsc

EMBED_DIM = 32
MAX_SIZE = 1000000000
VOCAB = 1000000
BATCH = 16384

NB = VOCAB // 16
K_IDX = np.float32(np.log10(2.0) / (9.0 / (VOCAB - 1)))

_POLY = (
    np.float32(-0.024825606495141983),
    np.float32(0.2668588161468506),
    np.float32(-1.2342631816864014),
    np.float32(3.2188327312469482),
    np.float32(-5.264110565185547),
    np.float32(6.065830230712891),
    np.float32(-3.028317451477051),
)


def _window_table() -> np.ndarray:
    b = np.logspace(0.0, np.log10(MAX_SIZE), num=VOCAB).astype(np.float32)
    idx = 16 * np.arange(NB - 1)[:, None] + np.arange(32)[None, :]
    return b[idx]


_BWIN = _window_table()

_NC, _NS, _L = 2, 16, 16
_NW = _NC * _NS
_BPW = BATCH // _NW
_GROUPS = _BPW // _L
_CHUNK = 128
_NCHUNK = _BPW // _CHUNK


def _sc_body(x_hbm, bwin_hbm, table_hbm, out_hbm,
             x_v, blk_v, win_v, idx_v, rows_v, sem):
    wid = lax.axis_index("s") * _NC + lax.axis_index("c")
    base = wid * _BPW

    pltpu.sync_copy(x_hbm.at[pl.ds(base, _BPW)], x_v)

    def guess_group(i, carry):
        x = x_v[pl.ds(i * _L, _L)]
        bits = lax.bitcast_convert_type(x, jnp.int32)
        e = (bits >> 23) - 127
        m = lax.bitcast_convert_type((bits & 0x7FFFFF) | 0x3F800000,
                                     jnp.float32)
        acc = jnp.full((_L,), _POLY[0], dtype=jnp.float32)
        for c in _POLY[1:]:
            acc = acc * m + c
        log2x = e.astype(jnp.float32) + acc
        g = (log2x * K_IDX).astype(jnp.int32) + 1
        blk = jnp.clip((g - 8) >> 4, 0, NB - 2)
        blk_v[pl.ds(i * _L, _L)] = blk
        return carry

    lax.fori_loop(0, _GROUPS, guess_group, 0, unroll=False)

    copies = [
        pltpu.async_copy(
            bwin_hbm.at[blk_v.at[pl.ds(j * _CHUNK, _CHUNK)]],
            win_v.at[pl.ds(j * _CHUNK, _CHUNK)],
            sem,
        )
        for j in range(_NCHUNK)
    ]
    for c in copies:
        c.wait()

    def count_group(i, carry):
        x = x_v[pl.ds(i * _L, _L)]
        blk = blk_v[pl.ds(i * _L, _L)]
        rows = i * _L + lax.iota(jnp.int32, _L)
        cnt = jnp.zeros((_L,), jnp.int32)
        for j in range(32):
            w = plsc.load_gather(win_v, [rows, jnp.full((_L,), j, jnp.int32)])
            cnt = cnt + jnp.where(w <= x, 1, 0)
        t = jnp.minimum(blk * 16 + cnt, VOCAB - 1)
        idx_v[pl.ds(i * _L, _L)] = t
        return carry

    lax.fori_loop(0, _GROUPS, count_group, 0, unroll=False)

    copies = [
        pltpu.async_copy(
            table_hbm.at[idx_v.at[pl.ds(j * _CHUNK, _CHUNK)]],
            rows_v.at[pl.ds(j * _CHUNK, _CHUNK)],
            sem,
        )
        for j in range(_NCHUNK)
    ]
    for c in copies:
        c.wait()

    pltpu.sync_copy(rows_v, out_hbm.at[pl.ds(base, _BPW)])


@jax.jit
def _embed(inputs, bwin, table):
    mesh = plsc.VectorSubcoreMesh(core_axis_name="c", subcore_axis_name="s")
    return pl.kernel(
        _sc_body,
        out_type=jax.ShapeDtypeStruct((BATCH, EMBED_DIM), jnp.float32),
        mesh=mesh,
        scratch_types=[
            pltpu.VMEM((_BPW,), jnp.float32),
            pltpu.VMEM((_BPW,), jnp.int32),
            pltpu.VMEM((_BPW, 32), jnp.float32),
            pltpu.VMEM((_BPW,), jnp.int32),
            pltpu.VMEM((_BPW, EMBED_DIM), jnp.float32),
            pltpu.SemaphoreType.DMA,
        ],
        compiler_params=pltpu.CompilerParams(
            needs_layout_passes=False, use_tc_tiling_on_sc=False),
    )(inputs, bwin, table)


def kernel(inputs, table):
    bwin = jnp.asarray(_BWIN)
    return _embed(inputs, bwin, table)


# speedup vs baseline: 16.7663x; 1.4287x over previous
import functools

import jax
import jax.numpy as jnp
import numpy as np
from jax import lax
from jax.experimental import pallas as pl
from jax.experimental.pallas import tpu as pltpu
from jax.experimental.pallas import tpu_sc as plsc

EMBED_DIM = 32
MAX_SIZE = 1000000000
VOCAB = 1000000
BATCH = 16384

NBW = 15625
K_IDX = np.float32(np.log10(2.0) / (9.0 / (VOCAB - 1)))

_POLY = (
    np.float32(-0.024825606495141983),
    np.float32(0.2668588161468506),
    np.float32(-1.2342631816864014),
    np.float32(3.2188327312469482),
    np.float32(-5.264110565185547),
    np.float32(6.065830230712891),
    np.float32(-3.028317451477051),
)


def _window_table() -> np.ndarray:
    b = np.logspace(0.0, np.log10(MAX_SIZE), num=VOCAB).astype(np.float32)
    bpad = np.concatenate(
        [b, np.full(64 * NBW + 128 - VOCAB, np.inf, dtype=np.float32)])
    idx = 64 * np.arange(NBW)[:, None] + np.arange(128)[None, :]
    return bpad[idx]


_BWIN = _window_table()

_NC, _NS, _L = 2, 16, 16
_NW = _NC * _NS
_BPW = BATCH // _NW
_GROUPS = _BPW // _L
_CHUNK = 128
_NCHUNK = _BPW // _CHUNK
_FCHUNK = 32
_BURST = 16


def _sc_body(x_hbm, bwin_hbm, table_hbm, out_hbm,
             x_v, blk_v, sub_v, win_v, q_v, s3_v, buf_v, rows_v, sem):
    wid = lax.axis_index("s") * _NC + lax.axis_index("c")
    base = wid * _BPW

    pltpu.sync_copy(x_hbm.at[pl.ds(base, _BPW)], x_v)

    def guess_group(i, carry):
        x = x_v[pl.ds(i * _L, _L)]
        bits = lax.bitcast_convert_type(x, jnp.int32)
        e = (bits >> 23) - 127
        m = lax.bitcast_convert_type((bits & 0x7FFFFF) | 0x3F800000,
                                     jnp.float32)
        acc = jnp.full((_L,), _POLY[0], dtype=jnp.float32)
        for c in _POLY[1:]:
            acc = acc * m + c
        log2x = e.astype(jnp.float32) + acc
        g = (log2x * K_IDX).astype(jnp.int32) + 1
        b0 = jnp.clip(g - 8, 0, VOCAB - 17)
        blk_v[pl.ds(i * _L, _L)] = b0 >> 6
        sub_v[pl.ds(i * _L, _L)] = b0
        return carry

    lax.fori_loop(0, _GROUPS, guess_group, 0, unroll=False)

    copies = [
        pltpu.async_copy(
            bwin_hbm.at[blk_v.at[pl.ds(j * _CHUNK, _CHUNK)]],
            win_v.at[pl.ds(j * _CHUNK, _CHUNK)],
            sem,
        )
        for j in range(_NCHUNK)
    ]
    for c in copies:
        c.wait()

    def count_group(i, carry):
        x = x_v[pl.ds(i * _L, _L)]
        b0 = sub_v[pl.ds(i * _L, _L)]
        s = b0 & 63
        rows = i * _L + lax.iota(jnp.int32, _L)
        cnt = jnp.zeros((_L,), jnp.int32)
        for j in range(17):
            w = plsc.load_gather(win_v, [rows, s + j])
            cnt = cnt + jnp.where(w <= x, 1, 0)
        t = jnp.minimum(b0 + cnt, VOCAB - 1)
        q_v[pl.ds(i * _L, _L)] = t >> 3
        s3_v[pl.ds(i * _L, _L)] = t & 7
        return carry

    lax.fori_loop(0, _GROUPS, count_group, 0, unroll=False)

    def fetch_chunk(fc, carry):
        e0 = fc * _FCHUNK
        descs = []
        for k in range(_FCHUNK // _BURST):
            qv = q_v[pl.ds(e0 + k * _BURST, _BURST)]
            for j in range(_BURST):
                q = qv[j]
                descs.append(pltpu.async_copy(
                    table_hbm.at[pl.ds(q * 8, 8)],
                    buf_v.at[k * _BURST + j],
                    sem,
                ))
        for dsc in descs:
            dsc.wait()

        for g in range(_FCHUNK // _L):
            e_loc = g * _L + lax.iota(jnp.int32, _L)
            s3 = s3_v[pl.ds(e0 + g * _L, _L)]
            for d in range(EMBED_DIM):
                dd = jnp.full((_L,), d, jnp.int32)
                w = plsc.load_gather(buf_v, [e_loc, s3, dd])
                plsc.store_scatter(rows_v, [e_loc >> 3, e_loc & 7, dd], w)

        pltpu.sync_copy(
            rows_v,
            out_hbm.at[pl.ds(wid * (_BPW // 8) + fc * (_FCHUNK // 8),
                             _FCHUNK // 8)])
        return carry

    lax.fori_loop(0, _BPW // _FCHUNK, fetch_chunk, 0, unroll=False)


@jax.jit
def _embed(inputs, bwin, table):
    mesh = plsc.VectorSubcoreMesh(core_axis_name="c", subcore_axis_name="s")
    out3 = pl.kernel(
        _sc_body,
        out_type=jax.ShapeDtypeStruct((BATCH // 8, 8, EMBED_DIM), jnp.float32),
        mesh=mesh,
        scratch_types=[
            pltpu.VMEM((_BPW,), jnp.float32),
            pltpu.VMEM((_BPW,), jnp.int32),
            pltpu.VMEM((_BPW,), jnp.int32),
            pltpu.VMEM((_BPW, 128), jnp.float32),
            pltpu.VMEM((_BPW,), jnp.int32),
            pltpu.VMEM((_BPW,), jnp.int32),
            pltpu.VMEM((_FCHUNK, 8, EMBED_DIM), jnp.float32),
            pltpu.VMEM((_FCHUNK // 8, 8, EMBED_DIM), jnp.float32),
            pltpu.SemaphoreType.DMA,
        ],
        compiler_params=pltpu.CompilerParams(
            needs_layout_passes=False, use_tc_tiling_on_sc=True),
    )(inputs, bwin, table)
    return jnp.reshape(out3, (BATCH, EMBED_DIM))


def kernel(inputs, table):
    bwin = jnp.asarray(_BWIN)
    return _embed(inputs, bwin, table)


# speedup vs baseline: 43.6462x; 2.6032x over previous
import functools

import jax
import jax.numpy as jnp
import numpy as np
from jax import lax
from jax.experimental import pallas as pl
from jax.experimental.pallas import tpu as pltpu
from jax.experimental.pallas import tpu_sc as plsc

EMBED_DIM = 32
MAX_SIZE = 1000000000
VOCAB = 1000000
BATCH = 16384

NBW = 15625
K_IDX = np.float32(np.log10(2.0) / (9.0 / (VOCAB - 1)))

_POLY = (
    np.float32(-0.024825606495141983),
    np.float32(0.2668588161468506),
    np.float32(-1.2342631816864014),
    np.float32(3.2188327312469482),
    np.float32(-5.264110565185547),
    np.float32(6.065830230712891),
    np.float32(-3.028317451477051),
)


def _window_table() -> np.ndarray:
    b = np.logspace(0.0, np.log10(MAX_SIZE), num=VOCAB).astype(np.float32)
    bpad = np.concatenate(
        [b, np.full(64 * NBW + 128 - VOCAB, np.inf, dtype=np.float32)])
    idx = 64 * np.arange(NBW)[:, None] + np.arange(128)[None, :]
    return bpad[idx]


_BWIN = _window_table()

_NC, _NS, _L = 2, 16, 16
_NW = _NC * _NS
_BPW = BATCH // _NW
_GROUPS = _BPW // _L
_CHUNK = 128
_NCHUNK = _BPW // _CHUNK
_RING = 8


def _sc_body(x_hbm, bwin_hbm, tabt_hbm, out_hbm,
             x_v, blk_v, sub_v, win_v, t_v, ring_v, obuf_v, sem):
    wid = lax.axis_index("s") * _NC + lax.axis_index("c")
    base = wid * _BPW

    pltpu.sync_copy(x_hbm.at[pl.ds(base, _BPW)], x_v)

    def guess_group(i, carry):
        x = x_v[pl.ds(i * _L, _L)]
        bits = lax.bitcast_convert_type(x, jnp.int32)
        e = (bits >> 23) - 127
        m = lax.bitcast_convert_type((bits & 0x7FFFFF) | 0x3F800000,
                                     jnp.float32)
        acc = jnp.full((_L,), _POLY[0], dtype=jnp.float32)
        for c in _POLY[1:]:
            acc = acc * m + c
        log2x = e.astype(jnp.float32) + acc
        g = (log2x * K_IDX).astype(jnp.int32) + 1
        b0 = jnp.clip(g - 8, 0, VOCAB - 17)
        blk_v[pl.ds(i * _L, _L)] = b0 >> 6
        sub_v[pl.ds(i * _L, _L)] = b0
        return carry

    lax.fori_loop(0, _GROUPS, guess_group, 0, unroll=False)

    copies = [
        pltpu.async_copy(
            bwin_hbm.at[blk_v.at[pl.ds(j * _CHUNK, _CHUNK)]],
            win_v.at[pl.ds(j * _CHUNK, _CHUNK)],
            sem,
        )
        for j in range(_NCHUNK)
    ]
    for c in copies:
        c.wait()

    def count_group(i, carry):
        x = x_v[pl.ds(i * _L, _L)]
        b0 = sub_v[pl.ds(i * _L, _L)]
        s = b0 & 63
        rows = i * _L + lax.iota(jnp.int32, _L)
        cnt = jnp.zeros((_L,), jnp.int32)
        for j in range(17):
            w = plsc.load_gather(win_v, [rows, s + j])
            cnt = cnt + jnp.where(w <= x, 1, 0)
        t = jnp.minimum(b0 + cnt, VOCAB - 1)
        t_v[pl.ds(i * _L, _L)] = t
        return carry

    lax.fori_loop(0, _GROUPS, count_group, 0, unroll=False)

    d_lo = lax.iota(jnp.int32, _L)
    d_hi = d_lo + _L

    def fetch(tv, j, slot):
        col = (tv[j] >> 7) * 128
        return pltpu.async_copy(
            tabt_hbm.at[pl.ds(0, EMBED_DIM), pl.ds(pl.multiple_of(col, 128),
                                                   128)],
            ring_v.at[slot],
            sem,
        )

    def extract(tv, j, slot, e_loc):
        off = jnp.full((_L,), tv[j] & 127, jnp.int32)
        ee = jnp.full((_L,), e_loc, jnp.int32)
        lo = plsc.load_gather(ring_v.at[slot], [d_lo, off])
        hi = plsc.load_gather(ring_v.at[slot], [d_hi, off])
        plsc.store_scatter(obuf_v, [d_lo, ee], lo)
        plsc.store_scatter(obuf_v, [d_hi, ee], hi)

    def chunk_loop(fc, carry):
        e0 = fc * _CHUNK
        for g in range(_CHUNK // _L):
            tv = t_v[pl.ds(e0 + g * _L, _L)]
            descs = []
            for j in range(_RING):
                descs.append(fetch(tv, j, j))
            for j in range(_L):
                descs[j].wait()
                extract(tv, j, j % _RING, g * _L + j)
                if j + _RING < _L:
                    descs.append(fetch(tv, j + _RING, (j + _RING) % _RING))
        pltpu.sync_copy(
            obuf_v,
            out_hbm.at[pl.ds(0, EMBED_DIM),
                       pl.ds(pl.multiple_of(base + e0, _CHUNK), _CHUNK)])
        return carry

    lax.fori_loop(0, _NCHUNK, chunk_loop, 0, unroll=False)


@jax.jit
def _embed(inputs, bwin, table):
    tabt = table.T
    mesh = plsc.VectorSubcoreMesh(core_axis_name="c", subcore_axis_name="s")
    out_t = pl.kernel(
        _sc_body,
        out_type=jax.ShapeDtypeStruct((EMBED_DIM, BATCH), jnp.float32),
        mesh=mesh,
        scratch_types=[
            pltpu.VMEM((_BPW,), jnp.float32),
            pltpu.VMEM((_BPW,), jnp.int32),
            pltpu.VMEM((_BPW,), jnp.int32),
            pltpu.VMEM((_BPW, 128), jnp.float32),
            pltpu.VMEM((_BPW,), jnp.int32),
            pltpu.VMEM((_RING, EMBED_DIM, 128), jnp.float32),
            pltpu.VMEM((EMBED_DIM, _CHUNK), jnp.float32),
            pltpu.SemaphoreType.DMA,
        ],
        compiler_params=pltpu.CompilerParams(
            needs_layout_passes=False, use_tc_tiling_on_sc=True),
    )(inputs, bwin, tabt)
    return out_t.T


def kernel(inputs, table):
    bwin = jnp.asarray(_BWIN)
    return _embed(inputs, bwin, table)
